# baseline (speedup 1.0000x reference)
import functools

import jax
import jax.numpy as jnp
from jax import lax
from jax.experimental import pallas as pl
from jax.experimental.pallas import tpu as pltpu
from jax.experimental.pallas import tpu_sc as plsc

N = 10000
E = 320000
DIM = 128
G = 128
L = 3

NC = 2
NS = 16
NW = NC * NS
CHUNK = 128
TPC_A = 156
TPC_B = 4
KBLK = 4
KBLK_LOG2 = 2
TOTCH = NS * (TPC_A + TPC_B)
EPAD = TOTCH * CHUNK
NPAD = 10112
ROWS_PT = NPAD // NS

_sc_mesh = plsc.VectorSubcoreMesh(core_axis_name="c", subcore_axis_name="s")


@functools.partial(
    pl.kernel,
    out_type=jax.ShapeDtypeStruct((NC * NPAD, DIM), jnp.float32),
    mesh=_sc_mesh,
    scratch_types=[
        pltpu.VMEM((2, KBLK * CHUNK), jnp.int32),
        pltpu.VMEM((2, KBLK, CHUNK), jnp.int32),
        pltpu.VMEM((2, CHUNK, DIM), jnp.float32),
        pltpu.VMEM_SHARED((NPAD, DIM), jnp.float32),
        pltpu.SemaphoreType.DMA((2,)),
        pltpu.SemaphoreType.DMA((2,)),
    ],
)
def _sc_agg(h_hbm, src_hbm, dst_hbm, zero_hbm, out_hbm,
            srcb_v, dstb_v, rows_v, agg_s, sem_r, sem_i):
    cid = lax.axis_index("c")
    sid = lax.axis_index("s")

    pltpu.sync_copy(zero_hbm, agg_s.at[pl.ds(sid * ROWS_PT, ROWS_PT)])

    tpc = lax.select(cid == 0, jnp.int32(TPC_A), jnp.int32(TPC_B))
    nblk = tpc // KBLK
    cbase = cid * (NS * TPC_A) + sid * tpc
    ebase = cbase * CHUNK

    def _idx_fetch(blk):
        bb = jnp.bitwise_and(blk, 1)
        off = pl.multiple_of(ebase + blk * (KBLK * CHUNK), KBLK * CHUNK)
        pltpu.async_copy(src_hbm.at[pl.ds(off, KBLK * CHUNK)],
                         srcb_v.at[bb], sem_i.at[bb])
        roff = pl.multiple_of(cbase + blk * KBLK, KBLK)
        pltpu.async_copy(dst_hbm.at[pl.ds(roff, KBLK)],
                         dstb_v.at[bb], sem_i.at[bb])

    def _idx_wait(blk):
        bb = jnp.bitwise_and(blk, 1)
        pltpu.make_async_copy(src_hbm.at[pl.ds(0, KBLK * CHUNK)],
                              srcb_v.at[bb], sem_i.at[bb]).wait()
        pltpu.make_async_copy(dst_hbm.at[pl.ds(0, KBLK)],
                              dstb_v.at[bb], sem_i.at[bb]).wait()

    def _gather_start(c, b):
        sb = jnp.bitwise_and(jnp.right_shift(c, KBLK_LOG2), 1)
        off = pl.multiple_of(jnp.bitwise_and(c, KBLK - 1) * CHUNK, CHUNK)
        pltpu.async_copy(h_hbm.at[srcb_v.at[sb, pl.ds(off, CHUNK)]],
                         rows_v.at[b], sem_r.at[b])

    def _gather_wait(b):
        pltpu.make_async_copy(h_hbm.at[srcb_v.at[0, pl.ds(0, CHUNK)]],
                              rows_v.at[b], sem_r.at[b]).wait()

    _idx_fetch(0)
    _idx_fetch(1)
    _idx_wait(0)
    _gather_start(0, 0)
    _gather_start(1, 1)
    plsc.subcore_barrier()

    def outer(blk, _):
        @pl.when(blk + 1 < nblk)
        def _():
            _idx_wait(blk + 1)

        def inner(pos, _):
            c = blk * KBLK + pos
            b = jnp.bitwise_and(pos, 1)
            _gather_wait(b)
            pltpu.sync_copy(rows_v.at[b],
                            agg_s.at[dstb_v.at[jnp.bitwise_and(blk, 1), pos]],
                            add=True)

            @pl.when(c + 2 < tpc)
            def _():
                _gather_start(c + 2, b)
            return ()

        lax.fori_loop(0, KBLK, inner, (), unroll=False)

        @pl.when(blk + 2 < nblk)
        def _():
            _idx_fetch(blk + 2)
        return ()

    lax.fori_loop(0, nblk, outer, (), unroll=False)
    plsc.subcore_barrier()

    pltpu.sync_copy(
        agg_s.at[pl.ds(sid * ROWS_PT, ROWS_PT)],
        out_hbm.at[pl.ds(cid * NPAD + sid * ROWS_PT, ROWS_PT)],
    )


def _mlp_body(h_ref, a0_ref, a1_ref, w1_ref, b1_ref, w2_ref, b2_ref,
              g_ref, be_ref, o_ref):
    z = h_ref[...] + a0_ref[...] + a1_ref[...]
    z = jax.lax.dot_general(z, w1_ref[...], (((1,), (0,)), ((), ())),
                            preferred_element_type=jnp.float32)
    z = jnp.maximum(z + b1_ref[...], 0.0)
    z = jax.lax.dot_general(z, w2_ref[...], (((1,), (0,)), ((), ())),
                            preferred_element_type=jnp.float32)
    z = jnp.maximum(z + b2_ref[...], 0.0)
    m = jnp.mean(z, axis=0, keepdims=True)
    v = jnp.mean(jnp.square(z - m), axis=0, keepdims=True)
    o_ref[...] = (z - m) / jnp.sqrt(v + 1e-5) * g_ref[...] + be_ref[...]


def _mlp_pool_body(h_ref, a0_ref, a1_ref, w1_ref, b1_ref, w2_ref, b2_ref,
                   g_ref, be_ref, batch_ref, o_ref):
    z = h_ref[...] + a0_ref[...] + a1_ref[...]
    z = jax.lax.dot_general(z, w1_ref[...], (((1,), (0,)), ((), ())),
                            preferred_element_type=jnp.float32)
    z = jnp.maximum(z + b1_ref[...], 0.0)
    z = jax.lax.dot_general(z, w2_ref[...], (((1,), (0,)), ((), ())),
                            preferred_element_type=jnp.float32)
    z = jnp.maximum(z + b2_ref[...], 0.0)
    m = jnp.mean(z, axis=0, keepdims=True)
    v = jnp.mean(jnp.square(z - m), axis=0, keepdims=True)
    z = (z - m) / jnp.sqrt(v + 1e-5) * g_ref[...] + be_ref[...]
    oh = (batch_ref[...] == lax.broadcasted_iota(jnp.int32, (N, G), 1))
    o_ref[...] = jax.lax.dot_general(
        oh.astype(jnp.float32), z, (((0,), (0,)), ((), ())),
        preferred_element_type=jnp.float32,
        precision=jax.lax.Precision.HIGHEST)


_mlp = pl.pallas_call(
    _mlp_body, out_shape=jax.ShapeDtypeStruct((N, DIM), jnp.float32))
_mlp_pool = pl.pallas_call(
    _mlp_pool_body, out_shape=jax.ShapeDtypeStruct((G, DIM), jnp.float32))


def kernel(x, edge_index, batch, W1, b1, W2, b2, gamma, beta):
    src = edge_index[0]
    dst = edge_index[1]
    pad = EPAD - E
    srcp = jnp.concatenate([src, jnp.zeros((pad,), jnp.int32)])
    dstp = jnp.concatenate([dst, jnp.full((pad,), N, jnp.int32)])
    dstp = dstp.reshape(TOTCH, CHUNK)
    zero_rows = jnp.zeros((ROWS_PT, DIM), jnp.float32)
    batch2 = batch.reshape(N, 1)

    h = x
    for i in range(L):
        parts = _sc_agg(h, srcp, dstp, zero_rows)
        a0 = parts[:N]
        a1 = parts[NPAD:NPAD + N]
        w1 = W1[i]
        w2 = W2[i]
        b1r = b1[i].reshape(1, DIM)
        b2r = b2[i].reshape(1, DIM)
        gr = gamma[i].reshape(1, DIM)
        br = beta[i].reshape(1, DIM)
        if i < L - 1:
            h = _mlp(h, a0, a1, w1, b1r, w2, b2r, gr, br)
        else:
            h = _mlp_pool(h, a0, a1, w1, b1r, w2, b2r, gr, br, batch2)
    return h


# speedup vs baseline: 1.0076x; 1.0076x over previous
import functools

import jax
import jax.numpy as jnp
from jax import lax
from jax.experimental import pallas as pl
from jax.experimental.pallas import tpu as pltpu
from jax.experimental.pallas import tpu_sc as plsc

N = 10000
E = 320000
DIM = 128
G = 128
L = 3

NC = 2
NS = 16
NW = NC * NS
CHUNK = 128
TPC_A = 152
TPC_B = 8
KBLK = 8
KBLK_LOG2 = 3
NBUF = 2
TOTCH = NS * (TPC_A + TPC_B)
EPAD = TOTCH * CHUNK
NPAD = 10112
ROWS_PT = NPAD // NS

_sc_mesh = plsc.VectorSubcoreMesh(core_axis_name="c", subcore_axis_name="s")


@functools.partial(
    pl.kernel,
    out_type=jax.ShapeDtypeStruct((NC * NPAD, DIM), jnp.float32),
    mesh=_sc_mesh,
    scratch_types=[
        pltpu.VMEM((2, KBLK * CHUNK), jnp.int32),
        pltpu.VMEM((2, KBLK, CHUNK), jnp.int32),
        pltpu.VMEM((NBUF, CHUNK, DIM), jnp.float32),
        pltpu.VMEM_SHARED((NPAD, DIM), jnp.float32),
        pltpu.SemaphoreType.DMA((NBUF,)),
        pltpu.SemaphoreType.DMA((2,)),
    ],
)
def _sc_agg(h_hbm, src_hbm, dst_hbm, zero_hbm, out_hbm,
            srcb_v, dstb_v, rows_v, agg_s, sem_r, sem_i):
    cid = lax.axis_index("c")
    sid = lax.axis_index("s")

    pltpu.sync_copy(zero_hbm, agg_s.at[pl.ds(sid * ROWS_PT, ROWS_PT)])

    tpc = lax.select(cid == 0, jnp.int32(TPC_A), jnp.int32(TPC_B))
    nblk = tpc // KBLK
    cbase = cid * (NS * TPC_A) + sid * tpc
    ebase = cbase * CHUNK

    def _idx_fetch(blk):
        bb = jnp.bitwise_and(blk, 1)
        off = pl.multiple_of(ebase + blk * (KBLK * CHUNK), KBLK * CHUNK)
        pltpu.async_copy(src_hbm.at[pl.ds(off, KBLK * CHUNK)],
                         srcb_v.at[bb], sem_i.at[bb])
        roff = pl.multiple_of(cbase + blk * KBLK, KBLK)
        pltpu.async_copy(dst_hbm.at[pl.ds(roff, KBLK)],
                         dstb_v.at[bb], sem_i.at[bb])

    def _idx_wait(blk):
        bb = jnp.bitwise_and(blk, 1)
        pltpu.make_async_copy(src_hbm.at[pl.ds(0, KBLK * CHUNK)],
                              srcb_v.at[bb], sem_i.at[bb]).wait()
        pltpu.make_async_copy(dst_hbm.at[pl.ds(0, KBLK)],
                              dstb_v.at[bb], sem_i.at[bb]).wait()

    def _gather_start(c, b):
        sb = jnp.bitwise_and(jnp.right_shift(c, KBLK_LOG2), 1)
        off = pl.multiple_of(jnp.bitwise_and(c, KBLK - 1) * CHUNK, CHUNK)
        pltpu.async_copy(h_hbm.at[srcb_v.at[sb, pl.ds(off, CHUNK)]],
                         rows_v.at[b], sem_r.at[b])

    def _gather_wait(b):
        pltpu.make_async_copy(h_hbm.at[srcb_v.at[0, pl.ds(0, CHUNK)]],
                              rows_v.at[b], sem_r.at[b]).wait()

    _idx_fetch(0)

    @pl.when(nblk > 1)
    def _():
        _idx_fetch(1)

    _idx_wait(0)
    for i in range(NBUF):
        _gather_start(jnp.int32(i), jnp.int32(i))
    plsc.subcore_barrier()

    def outer(blk, _):
        @pl.when(blk + 1 < nblk)
        def _():
            _idx_wait(blk + 1)

        def inner(pos, _):
            c = blk * KBLK + pos
            b = jnp.bitwise_and(c, NBUF - 1)
            _gather_wait(b)
            pltpu.sync_copy(rows_v.at[b],
                            agg_s.at[dstb_v.at[jnp.bitwise_and(blk, 1), pos]],
                            add=True)

            @pl.when(c + NBUF < tpc)
            def _():
                _gather_start(c + NBUF, b)
            return ()

        lax.fori_loop(0, KBLK, inner, (), unroll=False)

        @pl.when(blk + 2 < nblk)
        def _():
            _idx_fetch(blk + 2)
        return ()

    lax.fori_loop(0, nblk, outer, (), unroll=False)
    plsc.subcore_barrier()

    pltpu.sync_copy(
        agg_s.at[pl.ds(sid * ROWS_PT, ROWS_PT)],
        out_hbm.at[pl.ds(cid * NPAD + sid * ROWS_PT, ROWS_PT)],
    )


def _mlp_body(h_ref, a0_ref, a1_ref, w1_ref, b1_ref, w2_ref, b2_ref,
              g_ref, be_ref, o_ref):
    z = h_ref[...] + a0_ref[...] + a1_ref[...]
    z = jax.lax.dot_general(z, w1_ref[...], (((1,), (0,)), ((), ())),
                            preferred_element_type=jnp.float32)
    z = jnp.maximum(z + b1_ref[...], 0.0)
    z = jax.lax.dot_general(z, w2_ref[...], (((1,), (0,)), ((), ())),
                            preferred_element_type=jnp.float32)
    z = jnp.maximum(z + b2_ref[...], 0.0)
    m = jnp.mean(z, axis=0, keepdims=True)
    v = jnp.mean(jnp.square(z - m), axis=0, keepdims=True)
    o_ref[...] = (z - m) / jnp.sqrt(v + 1e-5) * g_ref[...] + be_ref[...]


def _mlp_pool_body(h_ref, a0_ref, a1_ref, w1_ref, b1_ref, w2_ref, b2_ref,
                   g_ref, be_ref, batch_ref, o_ref):
    z = h_ref[...] + a0_ref[...] + a1_ref[...]
    z = jax.lax.dot_general(z, w1_ref[...], (((1,), (0,)), ((), ())),
                            preferred_element_type=jnp.float32)
    z = jnp.maximum(z + b1_ref[...], 0.0)
    z = jax.lax.dot_general(z, w2_ref[...], (((1,), (0,)), ((), ())),
                            preferred_element_type=jnp.float32)
    z = jnp.maximum(z + b2_ref[...], 0.0)
    m = jnp.mean(z, axis=0, keepdims=True)
    v = jnp.mean(jnp.square(z - m), axis=0, keepdims=True)
    z = (z - m) / jnp.sqrt(v + 1e-5) * g_ref[...] + be_ref[...]
    oh = (batch_ref[...] == lax.broadcasted_iota(jnp.int32, (N, G), 1))
    o_ref[...] = jax.lax.dot_general(
        oh.astype(jnp.float32), z, (((0,), (0,)), ((), ())),
        preferred_element_type=jnp.float32,
        precision=jax.lax.Precision.HIGHEST)


_mlp = pl.pallas_call(
    _mlp_body, out_shape=jax.ShapeDtypeStruct((N, DIM), jnp.float32))
_mlp_pool = pl.pallas_call(
    _mlp_pool_body, out_shape=jax.ShapeDtypeStruct((G, DIM), jnp.float32))


def kernel(x, edge_index, batch, W1, b1, W2, b2, gamma, beta):
    src = edge_index[0]
    dst = edge_index[1]
    pad = EPAD - E
    srcp = jnp.concatenate([src, jnp.zeros((pad,), jnp.int32)])
    dstp = jnp.concatenate([dst, jnp.full((pad,), N, jnp.int32)])
    dstp = dstp.reshape(TOTCH, CHUNK)
    zero_rows = jnp.zeros((ROWS_PT, DIM), jnp.float32)
    batch2 = batch.reshape(N, 1)

    h = x
    for i in range(L):
        parts = _sc_agg(h, srcp, dstp, zero_rows)
        a0 = parts[:N]
        a1 = parts[NPAD:NPAD + N]
        w1 = W1[i]
        w2 = W2[i]
        b1r = b1[i].reshape(1, DIM)
        b2r = b2[i].reshape(1, DIM)
        gr = gamma[i].reshape(1, DIM)
        br = beta[i].reshape(1, DIM)
        if i < L - 1:
            h = _mlp(h, a0, a1, w1, b1r, w2, b2r, gr, br)
        else:
            h = _mlp_pool(h, a0, a1, w1, b1r, w2, b2r, gr, br, batch2)
    return h
